# baseline (speedup 1.0000x reference)
import functools

import jax
import jax.numpy as jnp
from jax import lax
from jax.experimental import pallas as pl
from jax.experimental.pallas import tpu as pltpu

N_ROWS = 131072
N_CLS = 16
LANES = 128
ROWS_PER_LANE_ROW = LANES // N_CLS
FLAT_ROWS = N_ROWS // ROWS_PER_LANE_ROW
NUM_BLOCKS = 16
BLOCK_ROWS = FLAT_ROWS // NUM_BLOCKS
_NEG_RATIO = 3.0
_EPS = 1e-06
_BISECT_ITERS = 40


def _body(pred_ref, gt_ref, g_ref, e_ref, p_ref, out_ref, scratch_ref, acc_ref):
    i = pl.program_id(0)

    pred = pred_ref[...]
    gt = gt_ref[...]
    log_p = jnp.maximum(jnp.log(pred), -100.0)
    log_1mp = jnp.maximum(jnp.log(1.0 - pred), -100.0)
    bce = -(gt * log_p + (1.0 - gt) * log_1mp)

    loss_rows = jnp.dot(bce, g_ref[...], preferred_element_type=jnp.float32)
    loss_rows = loss_rows * (1.0 / N_CLS)
    negflag = jnp.dot(gt, e_ref[...], preferred_element_type=jnp.float32)
    neg_rows = loss_rows * negflag

    pos_sum_blk = jnp.sum(loss_rows - neg_rows)
    neg_cnt_blk = jnp.sum(negflag)
    pack = jnp.dot(neg_rows, p_ref[0], preferred_element_type=jnp.float32)

    @pl.when(i == 0)
    def _init():
        scratch_ref[...] = pack
        acc_ref[0] = pos_sum_blk
        acc_ref[1] = neg_cnt_blk

    @pl.when(i > 0)
    def _accum():
        scratch_ref[...] = scratch_ref[...] + pack
        acc_ref[0] = acc_ref[0] + pos_sum_blk
        acc_ref[1] = acc_ref[1] + neg_cnt_blk

    @pl.when(i == NUM_BLOCKS - 1)
    def _finish():
        pos_sum = acc_ref[0]
        neg_total = acc_ref[1]
        pos_count = jnp.float32(N_ROWS) - neg_total
        k = jnp.minimum(neg_total, pos_count * _NEG_RATIO)

        s = scratch_ref[...]
        total_neg_sum = jnp.sum(s)
        nonzero_cnt = jnp.sum(jnp.where(s > 0.0, 1.0, 0.0))

        def topk_sum(_):
            hi0 = jnp.max(s)

            def step(_it, carry):
                lo, hi = carry
                mid = 0.5 * (lo + hi)
                cnt = jnp.sum(jnp.where(s > mid, 1.0, 0.0))
                take_hi = cnt > k
                return (jnp.where(take_hi, mid, lo), jnp.where(take_hi, hi, mid))

            _, t = lax.fori_loop(0, _BISECT_ITERS, step, (jnp.float32(0.0), hi0))
            cnt_t = jnp.sum(jnp.where(s > t, 1.0, 0.0))
            s_t = jnp.sum(jnp.where(s > t, s, 0.0))
            return s_t + t * jnp.maximum(k - cnt_t, 0.0)

        neg_top_sum = lax.cond(k >= nonzero_cnt, lambda _: total_neg_sum, topk_sum, 0.0)
        balanced = (pos_sum + neg_top_sum) / (pos_count + k + _EPS)
        out_ref[...] = jnp.full((8, 128), balanced, dtype=jnp.float32)


@functools.partial(jax.jit)
def _run(pred_flat, gt_flat, g_mat, e_mat, p_mat):
    out = pl.pallas_call(
        _body,
        grid=(NUM_BLOCKS,),
        in_specs=[
            pl.BlockSpec((BLOCK_ROWS, LANES), lambda i: (i, 0)),
            pl.BlockSpec((BLOCK_ROWS, LANES), lambda i: (i, 0)),
            pl.BlockSpec((LANES, LANES), lambda i: (0, 0)),
            pl.BlockSpec((LANES, LANES), lambda i: (0, 0)),
            pl.BlockSpec((1, LANES, LANES), lambda i: (i, 0, 0)),
        ],
        out_specs=pl.BlockSpec((8, 128), lambda i: (0, 0)),
        out_shape=jax.ShapeDtypeStruct((8, 128), jnp.float32),
        scratch_shapes=[
            pltpu.VMEM((BLOCK_ROWS, LANES), jnp.float32),
            pltpu.SMEM((2,), jnp.float32),
        ],
        compiler_params=pltpu.CompilerParams(
            dimension_semantics=("arbitrary",),
        ),
    )(pred_flat, gt_flat, g_mat, e_mat, p_mat)
    return out[0, 0]


def _constants():
    lane = jnp.arange(LANES)
    grp = jnp.arange(LANES)
    g_mat = ((lane[:, None] // N_CLS) == grp[None, :]).astype(jnp.float32)
    g_mat = g_mat * (grp[None, :] < ROWS_PER_LANE_ROW)
    e_mat = (lane[:, None] == (N_CLS * grp)[None, :]).astype(jnp.float32)
    e_mat = e_mat * (grp[None, :] < ROWS_PER_LANE_ROW)
    blk = jnp.arange(NUM_BLOCKS)
    p_mat = (
        (blk[:, None, None] * ROWS_PER_LANE_ROW + grp[None, :, None]) == lane[None, None, :]
    ).astype(jnp.float32)
    p_mat = p_mat * (grp[None, :, None] < ROWS_PER_LANE_ROW)
    return g_mat, e_mat, p_mat


def kernel(pred, gt):
    g_mat, e_mat, p_mat = _constants()
    pred_flat = pred.reshape(FLAT_ROWS, LANES)
    gt_flat = gt.reshape(FLAT_ROWS, LANES)
    return _run(pred_flat, gt_flat, g_mat, e_mat, p_mat)


# speedup vs baseline: 7.2382x; 7.2382x over previous
import functools

import jax
import jax.numpy as jnp
from jax import lax
from jax.experimental import pallas as pl
from jax.experimental.pallas import tpu as pltpu

N_ROWS = 131072
N_CLS = 16
NUM_BLOCKS = 16
BLOCK_N = N_ROWS // NUM_BLOCKS
_NEG_RATIO = 3.0
_EPS = 1e-06
_BISECT_ITERS = 40


def _body(pred_ref, gt_ref, out_ref, scratch_ref, acc_ref):
    i = pl.program_id(0)

    pred = pred_ref[...]
    gt = gt_ref[...]
    log_p = jnp.maximum(jnp.log(pred), -100.0)
    log_1mp = jnp.maximum(jnp.log(1.0 - pred), -100.0)
    bce = -(gt * log_p + (1.0 - gt) * log_1mp)

    loss = jnp.sum(bce, axis=0, keepdims=True) * (1.0 / N_CLS)
    negflag = gt_ref[0:1, :]
    neg = loss * negflag

    pos_sum_blk = jnp.sum(loss - neg)
    neg_cnt_blk = jnp.sum(negflag)
    neg_sum_blk = jnp.sum(neg)
    neg_nz_blk = jnp.sum(jnp.where(neg > 0.0, 1.0, 0.0))
    neg_max_blk = jnp.max(neg)
    scratch_ref[pl.ds(i, 1), :] = neg

    @pl.when(i == 0)
    def _init():
        acc_ref[0] = pos_sum_blk
        acc_ref[1] = neg_cnt_blk
        acc_ref[2] = neg_sum_blk
        acc_ref[3] = neg_nz_blk
        acc_ref[4] = neg_max_blk

    @pl.when(i > 0)
    def _accum():
        acc_ref[0] = acc_ref[0] + pos_sum_blk
        acc_ref[1] = acc_ref[1] + neg_cnt_blk
        acc_ref[2] = acc_ref[2] + neg_sum_blk
        acc_ref[3] = acc_ref[3] + neg_nz_blk
        acc_ref[4] = jnp.maximum(acc_ref[4], neg_max_blk)

    @pl.when(i == NUM_BLOCKS - 1)
    def _finish():
        pos_sum = acc_ref[0]
        neg_total = acc_ref[1]
        pos_count = jnp.float32(N_ROWS) - neg_total
        k = jnp.minimum(neg_total, pos_count * _NEG_RATIO)

        total_neg_sum = acc_ref[2]
        nonzero_cnt = acc_ref[3]

        def topk_sum(_):
            s = scratch_ref[...]
            hi0 = acc_ref[4]

            def step(_it, carry):
                lo, hi = carry
                mid = 0.5 * (lo + hi)
                cnt = jnp.sum(jnp.where(s > mid, 1.0, 0.0))
                take_hi = cnt > k
                return (jnp.where(take_hi, mid, lo), jnp.where(take_hi, hi, mid))

            _, t = lax.fori_loop(0, _BISECT_ITERS, step, (jnp.float32(0.0), hi0))
            cnt_t = jnp.sum(jnp.where(s > t, 1.0, 0.0))
            s_t = jnp.sum(jnp.where(s > t, s, 0.0))
            return s_t + t * jnp.maximum(k - cnt_t, 0.0)

        neg_top_sum = lax.cond(k >= nonzero_cnt, lambda _: total_neg_sum, topk_sum, 0.0)
        balanced = (pos_sum + neg_top_sum) / (pos_count + k + _EPS)
        out_ref[...] = jnp.full((8, 128), balanced, dtype=jnp.float32)


@functools.partial(jax.jit)
def _run(pred_t, gt_t):
    out = pl.pallas_call(
        _body,
        grid=(NUM_BLOCKS,),
        in_specs=[
            pl.BlockSpec((N_CLS, BLOCK_N), lambda i: (0, i)),
            pl.BlockSpec((N_CLS, BLOCK_N), lambda i: (0, i)),
        ],
        out_specs=pl.BlockSpec((8, 128), lambda i: (0, 0)),
        out_shape=jax.ShapeDtypeStruct((8, 128), jnp.float32),
        scratch_shapes=[
            pltpu.VMEM((NUM_BLOCKS, BLOCK_N), jnp.float32),
            pltpu.SMEM((8,), jnp.float32),
        ],
        compiler_params=pltpu.CompilerParams(
            dimension_semantics=("arbitrary",),
        ),
    )(pred_t, gt_t)
    return out[0, 0]


def kernel(pred, gt):
    return _run(pred.T, gt.T)


# speedup vs baseline: 9.9896x; 1.3801x over previous
import functools

import jax
import jax.numpy as jnp
from jax import lax
from jax.experimental import pallas as pl
from jax.experimental.pallas import tpu as pltpu

N_ROWS = 131072
N_CLS = 16
NUM_BLOCKS = 8
BLOCK_N = N_ROWS // NUM_BLOCKS
_NEG_RATIO = 3.0
_EPS = 1e-06
_BISECT_ITERS = 40


def _body(pred_ref, gt_ref, out_ref, scratch_ref, acc_ref):
    i = pl.program_id(0)

    pred = pred_ref[...]
    gt = gt_ref[...]
    log_p = jnp.maximum(jnp.log(pred), -100.0)
    log_1mp = jnp.maximum(jnp.log(1.0 - pred), -100.0)
    bce = gt * (log_1mp - log_p) - log_1mp

    loss = jnp.sum(bce, axis=0, keepdims=True) * (1.0 / N_CLS)
    negflag = gt_ref[0:1, :]
    neg = loss * negflag

    pos_sum_blk = jnp.sum(loss) - jnp.sum(neg)
    neg_cnt_blk = jnp.sum(negflag)
    neg_sum_blk = jnp.sum(neg)
    neg_nz_blk = jnp.sum(jnp.where(neg > 0.0, 1.0, 0.0))
    neg_max_blk = jnp.max(neg)
    scratch_ref[pl.ds(i, 1), :] = neg

    @pl.when(i == 0)
    def _init():
        acc_ref[0] = pos_sum_blk
        acc_ref[1] = neg_cnt_blk
        acc_ref[2] = neg_sum_blk
        acc_ref[3] = neg_nz_blk
        acc_ref[4] = neg_max_blk

    @pl.when(i > 0)
    def _accum():
        acc_ref[0] = acc_ref[0] + pos_sum_blk
        acc_ref[1] = acc_ref[1] + neg_cnt_blk
        acc_ref[2] = acc_ref[2] + neg_sum_blk
        acc_ref[3] = acc_ref[3] + neg_nz_blk
        acc_ref[4] = jnp.maximum(acc_ref[4], neg_max_blk)

    @pl.when(i == NUM_BLOCKS - 1)
    def _finish():
        pos_sum = acc_ref[0]
        neg_total = acc_ref[1]
        pos_count = jnp.float32(N_ROWS) - neg_total
        k = jnp.minimum(neg_total, pos_count * _NEG_RATIO)

        total_neg_sum = acc_ref[2]
        nonzero_cnt = acc_ref[3]

        def topk_sum(_):
            s = scratch_ref[...]
            hi0 = acc_ref[4]

            def step(_it, carry):
                lo, hi = carry
                mid = 0.5 * (lo + hi)
                cnt = jnp.sum(jnp.where(s > mid, 1.0, 0.0))
                take_hi = cnt > k
                return (jnp.where(take_hi, mid, lo), jnp.where(take_hi, hi, mid))

            _, t = lax.fori_loop(0, _BISECT_ITERS, step, (jnp.float32(0.0), hi0))
            cnt_t = jnp.sum(jnp.where(s > t, 1.0, 0.0))
            s_t = jnp.sum(jnp.where(s > t, s, 0.0))
            return s_t + t * jnp.maximum(k - cnt_t, 0.0)

        neg_top_sum = lax.cond(k >= nonzero_cnt, lambda _: total_neg_sum, topk_sum, 0.0)
        balanced = (pos_sum + neg_top_sum) / (pos_count + k + _EPS)
        out_ref[...] = jnp.full((8, 128), balanced, dtype=jnp.float32)


@functools.partial(jax.jit)
def _run(pred_t, gt_t):
    out = pl.pallas_call(
        _body,
        grid=(NUM_BLOCKS,),
        in_specs=[
            pl.BlockSpec((N_CLS, BLOCK_N), lambda i: (0, i)),
            pl.BlockSpec((N_CLS, BLOCK_N), lambda i: (0, i)),
        ],
        out_specs=pl.BlockSpec((8, 128), lambda i: (0, 0)),
        out_shape=jax.ShapeDtypeStruct((8, 128), jnp.float32),
        scratch_shapes=[
            pltpu.VMEM((NUM_BLOCKS, BLOCK_N), jnp.float32),
            pltpu.SMEM((8,), jnp.float32),
        ],
        compiler_params=pltpu.CompilerParams(
            dimension_semantics=("arbitrary",),
        ),
    )(pred_t, gt_t)
    return out[0, 0]


def kernel(pred, gt):
    return _run(pred.T, gt.T)


# speedup vs baseline: 11.1770x; 1.1189x over previous
import functools

import jax
import jax.numpy as jnp
from jax import lax
from jax.experimental import pallas as pl
from jax.experimental.pallas import tpu as pltpu

N_ROWS = 131072
N_CLS = 16
NUM_BLOCKS = 8
BLOCK_N = N_ROWS // NUM_BLOCKS
_NEG_RATIO = 3.0
_EPS = 1e-06
_BISECT_ITERS = 40


def _body(pred_ref, gt_ref, out_ref, scratch_ref, acc_ref):
    i = pl.program_id(0)

    pred = pred_ref[...]
    gt = gt_ref[...]
    p_hit = jnp.where(gt > 0.5, pred, 1.0 - pred)
    bce = jnp.maximum(jnp.log(p_hit), -100.0)

    w = jnp.full((8, N_CLS), -1.0 / N_CLS, dtype=jnp.float32)
    loss8 = jax.numpy.dot(w, bce, preferred_element_type=jnp.float32)
    loss = loss8[0:1, :]
    negflag = gt_ref[0:1, :]
    neg = loss * negflag

    pos_sum_blk = jnp.sum(loss) - jnp.sum(neg)
    neg_cnt_blk = jnp.sum(negflag)
    neg_sum_blk = jnp.sum(neg)
    neg_nz_blk = jnp.sum(jnp.where(neg > 0.0, 1.0, 0.0))
    neg_max_blk = jnp.max(neg)
    scratch_ref[pl.ds(i, 1), :] = neg

    @pl.when(i == 0)
    def _init():
        acc_ref[0] = pos_sum_blk
        acc_ref[1] = neg_cnt_blk
        acc_ref[2] = neg_sum_blk
        acc_ref[3] = neg_nz_blk
        acc_ref[4] = neg_max_blk

    @pl.when(i > 0)
    def _accum():
        acc_ref[0] = acc_ref[0] + pos_sum_blk
        acc_ref[1] = acc_ref[1] + neg_cnt_blk
        acc_ref[2] = acc_ref[2] + neg_sum_blk
        acc_ref[3] = acc_ref[3] + neg_nz_blk
        acc_ref[4] = jnp.maximum(acc_ref[4], neg_max_blk)

    @pl.when(i == NUM_BLOCKS - 1)
    def _finish():
        pos_sum = acc_ref[0]
        neg_total = acc_ref[1]
        pos_count = jnp.float32(N_ROWS) - neg_total
        k = jnp.minimum(neg_total, pos_count * _NEG_RATIO)

        total_neg_sum = acc_ref[2]
        nonzero_cnt = acc_ref[3]

        def topk_sum(_):
            s = scratch_ref[...]
            hi0 = acc_ref[4]

            def step(_it, carry):
                lo, hi = carry
                mid = 0.5 * (lo + hi)
                cnt = jnp.sum(jnp.where(s > mid, 1.0, 0.0))
                take_hi = cnt > k
                return (jnp.where(take_hi, mid, lo), jnp.where(take_hi, hi, mid))

            _, t = lax.fori_loop(0, _BISECT_ITERS, step, (jnp.float32(0.0), hi0))
            cnt_t = jnp.sum(jnp.where(s > t, 1.0, 0.0))
            s_t = jnp.sum(jnp.where(s > t, s, 0.0))
            return s_t + t * jnp.maximum(k - cnt_t, 0.0)

        neg_top_sum = lax.cond(k >= nonzero_cnt, lambda _: total_neg_sum, topk_sum, 0.0)
        balanced = (pos_sum + neg_top_sum) / (pos_count + k + _EPS)
        out_ref[...] = jnp.full((8, 128), balanced, dtype=jnp.float32)


@functools.partial(jax.jit)
def _run(pred_t, gt_t):
    out = pl.pallas_call(
        _body,
        grid=(NUM_BLOCKS,),
        in_specs=[
            pl.BlockSpec((N_CLS, BLOCK_N), lambda i: (0, i)),
            pl.BlockSpec((N_CLS, BLOCK_N), lambda i: (0, i)),
        ],
        out_specs=pl.BlockSpec((8, 128), lambda i: (0, 0)),
        out_shape=jax.ShapeDtypeStruct((8, 128), jnp.float32),
        scratch_shapes=[
            pltpu.VMEM((NUM_BLOCKS, BLOCK_N), jnp.float32),
            pltpu.SMEM((8,), jnp.float32),
        ],
        compiler_params=pltpu.CompilerParams(
            dimension_semantics=("arbitrary",),
        ),
    )(pred_t, gt_t)
    return out[0, 0]


def kernel(pred, gt):
    return _run(pred.T, gt.T)


# speedup vs baseline: 13.6400x; 1.2204x over previous
import functools

import jax
import jax.numpy as jnp
from jax import lax
from jax.experimental import pallas as pl
from jax.experimental.pallas import tpu as pltpu

N_ROWS = 131072
N_CLS = 16
NUM_BLOCKS = 2
BLOCK_N = N_ROWS // NUM_BLOCKS
_NEG_RATIO = 3.0
_EPS = 1e-06
_BISECT_ITERS = 40


def _body(pred_ref, gt_ref, out_ref, scratch_ref, acc_ref):
    i = pl.program_id(0)

    pred = pred_ref[...]
    gt = gt_ref[...]
    p_hit = jnp.where(gt > 0.5, pred, 1.0 - pred)
    bce = jnp.maximum(jnp.log(p_hit), -100.0)

    w = jnp.full((8, N_CLS), -1.0 / N_CLS, dtype=jnp.float32)
    loss8 = jax.numpy.dot(w, bce, preferred_element_type=jnp.float32)
    loss = loss8[0:1, :]
    negflag = gt_ref[0:1, :]
    neg = loss * negflag

    pos_sum_blk = jnp.sum(loss) - jnp.sum(neg)
    neg_cnt_blk = jnp.sum(negflag)
    neg_sum_blk = jnp.sum(neg)
    neg_nz_blk = jnp.sum(jnp.where(neg > 0.0, 1.0, 0.0))
    neg_max_blk = jnp.max(neg)
    scratch_ref[pl.ds(i, 1), :] = neg

    @pl.when(i == 0)
    def _init():
        acc_ref[0] = pos_sum_blk
        acc_ref[1] = neg_cnt_blk
        acc_ref[2] = neg_sum_blk
        acc_ref[3] = neg_nz_blk
        acc_ref[4] = neg_max_blk

    @pl.when(i > 0)
    def _accum():
        acc_ref[0] = acc_ref[0] + pos_sum_blk
        acc_ref[1] = acc_ref[1] + neg_cnt_blk
        acc_ref[2] = acc_ref[2] + neg_sum_blk
        acc_ref[3] = acc_ref[3] + neg_nz_blk
        acc_ref[4] = jnp.maximum(acc_ref[4], neg_max_blk)

    @pl.when(i == NUM_BLOCKS - 1)
    def _finish():
        pos_sum = acc_ref[0]
        neg_total = acc_ref[1]
        pos_count = jnp.float32(N_ROWS) - neg_total
        k = jnp.minimum(neg_total, pos_count * _NEG_RATIO)

        total_neg_sum = acc_ref[2]
        nonzero_cnt = acc_ref[3]

        def topk_sum(_):
            s = scratch_ref[...]
            hi0 = acc_ref[4]

            def step(_it, carry):
                lo, hi = carry
                mid = 0.5 * (lo + hi)
                cnt = jnp.sum(jnp.where(s > mid, 1.0, 0.0))
                take_hi = cnt > k
                return (jnp.where(take_hi, mid, lo), jnp.where(take_hi, hi, mid))

            _, t = lax.fori_loop(0, _BISECT_ITERS, step, (jnp.float32(0.0), hi0))
            cnt_t = jnp.sum(jnp.where(s > t, 1.0, 0.0))
            s_t = jnp.sum(jnp.where(s > t, s, 0.0))
            return s_t + t * jnp.maximum(k - cnt_t, 0.0)

        neg_top_sum = lax.cond(k >= nonzero_cnt, lambda _: total_neg_sum, topk_sum, 0.0)
        balanced = (pos_sum + neg_top_sum) / (pos_count + k + _EPS)
        out_ref[...] = jnp.full((8, 128), balanced, dtype=jnp.float32)


@functools.partial(jax.jit)
def _run(pred_t, gt_t):
    out = pl.pallas_call(
        _body,
        grid=(NUM_BLOCKS,),
        in_specs=[
            pl.BlockSpec((N_CLS, BLOCK_N), lambda i: (0, i)),
            pl.BlockSpec((N_CLS, BLOCK_N), lambda i: (0, i)),
        ],
        out_specs=pl.BlockSpec((8, 128), lambda i: (0, 0)),
        out_shape=jax.ShapeDtypeStruct((8, 128), jnp.float32),
        scratch_shapes=[
            pltpu.VMEM((NUM_BLOCKS, BLOCK_N), jnp.float32),
            pltpu.SMEM((8,), jnp.float32),
        ],
        compiler_params=pltpu.CompilerParams(
            dimension_semantics=("arbitrary",),
        ),
    )(pred_t, gt_t)
    return out[0, 0]


def kernel(pred, gt):
    return _run(pred.T, gt.T)
